# baseline (speedup 1.0000x reference)
import functools

import jax
import jax.numpy as jnp
import numpy as np
from jax import lax
from jax.experimental import pallas as pl
from jax.experimental.pallas import tpu as pltpu
from jax.experimental.pallas import tpu_sc as plsc

N = 10000
E = 320000
D = 128
DE = 16
D_OUT = 128

NC = 2
NS = 16
NW = NC * NS

C = 80
NCHUNK0 = 124
NCHUNK1 = 126
RPW = 624
ZR = 16

_mesh = plsc.VectorSubcoreMesh(core_axis_name="c", subcore_axis_name="s")

_EIDX = np.arange(D).reshape(4, 2, 16)
_EPERM = np.concatenate([_EIDX[:, 0, :].ravel(), _EIDX[:, 1, :].ravel()])


@functools.partial(
    pl.kernel,
    out_type=jax.ShapeDtypeStruct((NC * N, D), jnp.float32),
    mesh=_mesh,
    scratch_types=[
        pltpu.VMEM((C, D), jnp.float32),
        pltpu.VMEM((C, D), jnp.float32),
        pltpu.VMEM((C, D // 2), jnp.int32),
        pltpu.VMEM((C, D // 2), jnp.int32),
        pltpu.VMEM((C,), jnp.int32),
        pltpu.VMEM((C,), jnp.int32),
        pltpu.VMEM((C,), jnp.int32),
        pltpu.VMEM((C,), jnp.int32),
        pltpu.VMEM((ZR, D), jnp.float32),
        pltpu.VMEM_SHARED((N, D), jnp.float32),
        pltpu.SemaphoreType.DMA,
        pltpu.SemaphoreType.DMA,
        pltpu.SemaphoreType.DMA,
        pltpu.SemaphoreType.DMA,
        pltpu.SemaphoreType.DMA,
        pltpu.SemaphoreType.DMA,
        pltpu.SemaphoreType.DMA,
        pltpu.SemaphoreType.DMA,
        pltpu.SemaphoreType.DMA,
        pltpu.SemaphoreType.DMA,
        pltpu.SemaphoreType.DMA,
    ],
)
def _sc_aggregate(src_hbm, dst_hbm, e_hbm, h_hbm, out_hbm,
                  rows0, rows1, e0, e1, src0, src1, dst0, dst1, zbuf, agg_sh,
                  gsem0, gsem1, esem0, esem1, rsem0, rsem1, dsem0, dsem1,
                  ssem0, ssem1, zsem):
    cid = lax.axis_index("c")
    sid = lax.axis_index("s")
    ebase = jnp.where(cid == 0, sid * (NCHUNK0 * C),
                      NS * (NCHUNK0 * C) + sid * (NCHUNK1 * C))
    nchunk = jnp.where(cid == 0, NCHUNK0, NCHUNK1)

    rows = (rows0, rows1)
    ebufs = (e0, e1)
    srcs = (src0, src1)
    dsts = (dst0, dst1)
    gsems = (gsem0, gsem1)
    esems = (esem0, esem1)
    rsems = (rsem0, rsem1)
    dsems = (dsem0, dsem1)
    ssems = (ssem0, ssem1)

    row_base = sid * RPW
    n_row_chunks = jnp.where(sid == NS - 1, (N - (NS - 1) * RPW) // ZR,
                             RPW // ZR)

    def _issue_idx(i, b):
        pltpu.async_copy(src_hbm.at[pl.ds(ebase + i * C, C)], srcs[b],
                         rsems[b])
        pltpu.async_copy(dst_hbm.at[pl.ds(ebase + i * C, C)], dsts[b],
                         dsems[b])

    def _issue_gather_e(i, b):
        pltpu.async_copy(h_hbm.at[srcs[b]], rows[b], gsems[b])
        pltpu.async_copy(e_hbm.at[pl.ds(ebase + i * C, C)], ebufs[b], esems[b])

    def _drain(sem, buf, dummy_src):
        pltpu.make_async_copy(dummy_src, buf, sem).wait()

    def _wait_idx(b):
        _drain(rsems[b], srcs[b], src_hbm.at[pl.ds(0, C)])
        _drain(dsems[b], dsts[b], dst_hbm.at[pl.ds(0, C)])

    def _wait_gather_e(b):
        _drain(gsems[b], rows[b], e_hbm.at[pl.ds(0, C)])
        _drain(esems[b], ebufs[b], e_hbm.at[pl.ds(0, C)])

    def _compute(b):
        rv, ev = rows[b], ebufs[b]

        def _rows4(r4, _):
            r = r4 * 4
            for u in range(4):
                for j in range(D // 32):
                    ew = ev[r + u, pl.ds(j * 16, 16)]
                    lo = lax.bitcast_convert_type(lax.shift_left(ew, 16),
                                                  jnp.float32)
                    hi = lax.bitcast_convert_type(ew & jnp.int32(-65536),
                                                  jnp.float32)
                    sl0 = pl.ds(j * 32, 16)
                    sl1 = pl.ds(j * 32 + 16, 16)
                    rv[r + u, sl0] = jnp.maximum(rv[r + u, sl0] + lo, 0.0)
                    rv[r + u, sl1] = jnp.maximum(rv[r + u, sl1] + hi, 0.0)
            return 0

        lax.fori_loop(0, C // 4, _rows4, 0)

    _issue_idx(0, 0)

    def _zero_row(r, _):
        for j in range(D // 16):
            zbuf[r, pl.ds(j * 16, 16)] = jnp.zeros((16,), jnp.float32)
        return 0

    lax.fori_loop(0, ZR, _zero_row, 0)

    _wait_idx(0)
    _issue_gather_e(0, 0)
    _issue_idx(1, 1)

    def _zero_issue(k, _):
        pltpu.async_copy(zbuf, agg_sh.at[pl.ds(row_base + k * ZR, ZR)], zsem)
        return 0

    lax.fori_loop(0, n_row_chunks, _zero_issue, 0)

    def _zero_drain(k, _):
        pltpu.make_async_copy(zbuf, agg_sh.at[pl.ds(row_base + k * ZR, ZR)],
                              zsem).wait()
        return 0

    lax.fori_loop(0, n_row_chunks, _zero_drain, 0)
    plsc.subcore_barrier()

    def _pair(t, _):
        for b in range(2):
            i = 2 * t + b
            nb = 1 - b
            i1 = lax.rem(i + 1, nchunk)
            i2 = lax.rem(i + 2, nchunk)
            _wait_idx(nb)

            @pl.when(i >= 1)
            def _retire():
                _drain(ssems[nb], rows[nb], e_hbm.at[pl.ds(0, C)])

            _issue_gather_e(i1, nb)
            _wait_gather_e(b)
            _compute(b)
            pltpu.async_copy(rows[b], agg_sh.at[dsts[b]], ssems[b], add=True)
            _issue_idx(i2, b)
        return 0

    lax.fori_loop(0, nchunk // 2, _pair, 0)

    _wait_gather_e(0)
    _wait_idx(1)
    _drain(ssems[1], rows[1], e_hbm.at[pl.ds(0, C)])
    plsc.subcore_barrier()

    def _out_issue(k, _):
        r0 = row_base + k * ZR
        pltpu.async_copy(agg_sh.at[pl.ds(r0, ZR)],
                         out_hbm.at[pl.ds(cid * N + r0, ZR)], zsem)
        return 0

    lax.fori_loop(0, n_row_chunks, _out_issue, 0)

    def _out_drain(k, _):
        r0 = row_base + k * ZR
        pltpu.make_async_copy(agg_sh.at[pl.ds(r0, ZR)],
                              out_hbm.at[pl.ds(cid * N + r0, ZR)],
                              zsem).wait()
        return 0

    lax.fori_loop(0, n_row_chunks, _out_drain, 0)


BE = 2000
BN = 2000


def _to_bf16_bits(x):
    b = jax.lax.bitcast_convert_type(x, jnp.int32)
    return jax.lax.shift_right_logical(
        b + 0x7FFF + (jax.lax.shift_right_logical(b, 16) & 1), 16)


def _enc_body(a_ref, w_ref, b_ref, o_ref):
    t = (
        jnp.dot(a_ref[...], w_ref[...], preferred_element_type=jnp.float32)
        + b_ref[...]
    )
    lo = _to_bf16_bits(t[:, : D // 2])
    hi = _to_bf16_bits(t[:, D // 2:])
    o_ref[...] = jax.lax.shift_left(hi, 16) | lo


_encode = pl.pallas_call(
    _enc_body,
    grid=(E // BE,),
    in_specs=[
        pl.BlockSpec((BE, DE), lambda i: (i, 0)),
        pl.BlockSpec((DE, D), lambda i: (0, 0)),
        pl.BlockSpec((1, D), lambda i: (0, 0)),
    ],
    out_specs=pl.BlockSpec((BE, D // 2), lambda i: (i, 0)),
    out_shape=jax.ShapeDtypeStruct((E, D // 2), jnp.int32),
)


def _layer_body(eps_ref, h_ref, p0_ref, p1_ref, w1_ref, b1_ref, w2_ref,
                b2_ref, o_ref):
    h = h_ref[...]
    t = (1.0 + eps_ref[0]) * h + p0_ref[...] + p1_ref[...]
    t = jnp.maximum(
        jnp.dot(t, w1_ref[...], preferred_element_type=jnp.float32)
        + b1_ref[...], 0.0)
    t = jnp.dot(t, w2_ref[...], preferred_element_type=jnp.float32) + b2_ref[...]
    o_ref[...] = h + jnp.maximum(t, 0.0)


_layer = pl.pallas_call(
    _layer_body,
    grid=(N // BN,),
    in_specs=[
        pl.BlockSpec(memory_space=pltpu.SMEM),
        pl.BlockSpec((BN, D), lambda i: (i, 0)),
        pl.BlockSpec((BN, D), lambda i: (i, 0)),
        pl.BlockSpec((BN, D), lambda i: (i + N // BN, 0)),
        pl.BlockSpec((D, D), lambda i: (0, 0)),
        pl.BlockSpec((1, D), lambda i: (0, 0)),
        pl.BlockSpec((D, D), lambda i: (0, 0)),
        pl.BlockSpec((1, D), lambda i: (0, 0)),
    ],
    out_specs=pl.BlockSpec((BN, D), lambda i: (i, 0)),
    out_shape=jax.ShapeDtypeStruct((N, D), jnp.float32),
)


def _layer_head_body(eps_ref, h_ref, p0_ref, p1_ref, w1_ref, b1_ref, w2_ref,
                     b2_ref, wh_ref, bh_ref, o_ref):
    h = h_ref[...]
    t = (1.0 + eps_ref[0]) * h + p0_ref[...] + p1_ref[...]
    t = jnp.maximum(
        jnp.dot(t, w1_ref[...], preferred_element_type=jnp.float32)
        + b1_ref[...], 0.0)
    t = jnp.dot(t, w2_ref[...], preferred_element_type=jnp.float32) + b2_ref[...]
    h = h + jnp.maximum(t, 0.0)
    o_ref[...] = (
        jnp.dot(h, wh_ref[...], preferred_element_type=jnp.float32)
        + bh_ref[...]
    )


_layer_head = pl.pallas_call(
    _layer_head_body,
    grid=(N // BN,),
    in_specs=[
        pl.BlockSpec(memory_space=pltpu.SMEM),
        pl.BlockSpec((BN, D), lambda i: (i, 0)),
        pl.BlockSpec((BN, D), lambda i: (i, 0)),
        pl.BlockSpec((BN, D), lambda i: (i + N // BN, 0)),
        pl.BlockSpec((D, D), lambda i: (0, 0)),
        pl.BlockSpec((1, D), lambda i: (0, 0)),
        pl.BlockSpec((D, D), lambda i: (0, 0)),
        pl.BlockSpec((1, D), lambda i: (0, 0)),
        pl.BlockSpec((D, D), lambda i: (0, 0)),
        pl.BlockSpec((1, D), lambda i: (0, 0)),
    ],
    out_specs=pl.BlockSpec((BN, D), lambda i: (i, 0)),
    out_shape=jax.ShapeDtypeStruct((N, D_OUT), jnp.float32),
)


def kernel(x, edge_index, edge_attr, W_edge, b_edge, eps, W1s, b1s, W2s, b2s,
           W_head, b_head):
    src = edge_index[0]
    dst = edge_index[1]
    e = _encode(edge_attr, W_edge[:, _EPERM], b_edge[_EPERM].reshape(1, D))
    h = x
    for l in range(2):
        parts = _sc_aggregate(src, dst, e, h)
        h = _layer(eps[l].reshape(1), h, parts, parts,
                   W1s[l], b1s[l].reshape(1, D),
                   W2s[l], b2s[l].reshape(1, D))
    parts = _sc_aggregate(src, dst, e, h)
    return _layer_head(eps[2].reshape(1), h, parts, parts,
                       W1s[2], b1s[2].reshape(1, D),
                       W2s[2], b2s[2].reshape(1, D),
                       W_head, b_head.reshape(1, D))


# speedup vs baseline: 1.0135x; 1.0135x over previous
import functools

import jax
import jax.numpy as jnp
import numpy as np
from jax import lax
from jax.experimental import pallas as pl
from jax.experimental.pallas import tpu as pltpu
from jax.experimental.pallas import tpu_sc as plsc

N = 10000
E = 320000
D = 128
DE = 16
D_OUT = 128

NC = 2
NS = 16
NW = NC * NS

C = 80
NCHUNK0 = 124
NCHUNK1 = 126
RPW = 624
ZR = 16

_mesh = plsc.VectorSubcoreMesh(core_axis_name="c", subcore_axis_name="s")


@functools.partial(
    pl.kernel,
    out_type=jax.ShapeDtypeStruct((NC * N, D), jnp.float32),
    mesh=_mesh,
    scratch_types=[
        pltpu.VMEM((C, D), jnp.float32),
        pltpu.VMEM((C, D), jnp.float32),
        pltpu.VMEM((C, D), jnp.float32),
        pltpu.VMEM((C, D), jnp.float32),
        pltpu.VMEM((C,), jnp.int32),
        pltpu.VMEM((C,), jnp.int32),
        pltpu.VMEM((C,), jnp.int32),
        pltpu.VMEM((C,), jnp.int32),
        pltpu.VMEM((ZR, D), jnp.float32),
        pltpu.VMEM_SHARED((N, D), jnp.float32),
        pltpu.SemaphoreType.DMA,
        pltpu.SemaphoreType.DMA,
        pltpu.SemaphoreType.DMA,
        pltpu.SemaphoreType.DMA,
        pltpu.SemaphoreType.DMA,
        pltpu.SemaphoreType.DMA,
        pltpu.SemaphoreType.DMA,
        pltpu.SemaphoreType.DMA,
        pltpu.SemaphoreType.DMA,
        pltpu.SemaphoreType.DMA,
        pltpu.SemaphoreType.DMA,
    ],
)
def _sc_aggregate(src_hbm, dst_hbm, e_hbm, h_hbm, out_hbm,
                  rows0, rows1, e0, e1, src0, src1, dst0, dst1, zbuf, agg_sh,
                  gsem0, gsem1, esem0, esem1, rsem0, rsem1, dsem0, dsem1,
                  ssem0, ssem1, zsem):
    cid = lax.axis_index("c")
    sid = lax.axis_index("s")
    ebase = jnp.where(cid == 0, sid * (NCHUNK0 * C),
                      NS * (NCHUNK0 * C) + sid * (NCHUNK1 * C))
    nchunk = jnp.where(cid == 0, NCHUNK0, NCHUNK1)

    rows = (rows0, rows1)
    ebufs = (e0, e1)
    srcs = (src0, src1)
    dsts = (dst0, dst1)
    gsems = (gsem0, gsem1)
    esems = (esem0, esem1)
    rsems = (rsem0, rsem1)
    dsems = (dsem0, dsem1)
    ssems = (ssem0, ssem1)

    row_base = sid * RPW
    n_row_chunks = jnp.where(sid == NS - 1, (N - (NS - 1) * RPW) // ZR,
                             RPW // ZR)

    def _issue_idx(i, b):
        pltpu.async_copy(src_hbm.at[pl.ds(ebase + i * C, C)], srcs[b],
                         rsems[b])
        pltpu.async_copy(dst_hbm.at[pl.ds(ebase + i * C, C)], dsts[b],
                         dsems[b])

    def _issue_gather_e(i, b):
        pltpu.async_copy(h_hbm.at[srcs[b]], rows[b], gsems[b])
        pltpu.async_copy(e_hbm.at[pl.ds(ebase + i * C, C)], ebufs[b], esems[b])

    def _drain(sem, buf, dummy_src):
        pltpu.make_async_copy(dummy_src, buf, sem).wait()

    def _wait_idx(b):
        _drain(rsems[b], srcs[b], src_hbm.at[pl.ds(0, C)])
        _drain(dsems[b], dsts[b], dst_hbm.at[pl.ds(0, C)])

    def _wait_gather_e(b):
        _drain(gsems[b], rows[b], e_hbm.at[pl.ds(0, C)])
        _drain(esems[b], ebufs[b], e_hbm.at[pl.ds(0, C)])

    def _compute(b):
        rv, ev = rows[b], ebufs[b]

        def _rows4(r4, _):
            r = r4 * 4
            for u in range(4):
                for j in range(D // 16):
                    sl = pl.ds(j * 16, 16)
                    rv[r + u, sl] = jnp.maximum(rv[r + u, sl] + ev[r + u, sl],
                                                0.0)
            return 0

        lax.fori_loop(0, C // 4, _rows4, 0)

    _issue_idx(0, 0)

    def _zero_row(r, _):
        for j in range(D // 16):
            zbuf[r, pl.ds(j * 16, 16)] = jnp.zeros((16,), jnp.float32)
        return 0

    lax.fori_loop(0, ZR, _zero_row, 0)

    _wait_idx(0)
    _issue_gather_e(0, 0)
    _issue_idx(1, 1)

    def _zero_issue(k, _):
        pltpu.async_copy(zbuf, agg_sh.at[pl.ds(row_base + k * ZR, ZR)], zsem)
        return 0

    lax.fori_loop(0, n_row_chunks, _zero_issue, 0)

    def _zero_drain(k, _):
        pltpu.make_async_copy(zbuf, agg_sh.at[pl.ds(row_base + k * ZR, ZR)],
                              zsem).wait()
        return 0

    lax.fori_loop(0, n_row_chunks, _zero_drain, 0)
    plsc.subcore_barrier()

    def _pair(t, _):
        for b in range(2):
            i = 2 * t + b
            nb = 1 - b
            i1 = lax.rem(i + 1, nchunk)
            i2 = lax.rem(i + 2, nchunk)
            _wait_idx(nb)

            @pl.when(i >= 1)
            def _retire():
                _drain(ssems[nb], rows[nb], e_hbm.at[pl.ds(0, C)])

            _issue_gather_e(i1, nb)
            _wait_gather_e(b)
            _compute(b)
            pltpu.async_copy(rows[b], agg_sh.at[dsts[b]], ssems[b], add=True)
            _issue_idx(i2, b)
        return 0

    lax.fori_loop(0, nchunk // 2, _pair, 0)

    _wait_gather_e(0)
    _wait_idx(1)
    _drain(ssems[1], rows[1], e_hbm.at[pl.ds(0, C)])
    plsc.subcore_barrier()

    def _out_issue(k, _):
        r0 = row_base + k * ZR
        pltpu.async_copy(agg_sh.at[pl.ds(r0, ZR)],
                         out_hbm.at[pl.ds(cid * N + r0, ZR)], zsem)
        return 0

    lax.fori_loop(0, n_row_chunks, _out_issue, 0)

    def _out_drain(k, _):
        r0 = row_base + k * ZR
        pltpu.make_async_copy(agg_sh.at[pl.ds(r0, ZR)],
                              out_hbm.at[pl.ds(cid * N + r0, ZR)],
                              zsem).wait()
        return 0

    lax.fori_loop(0, n_row_chunks, _out_drain, 0)


BE = 2000
BN = 2000


def _enc_body(a_ref, w_ref, b_ref, o_ref):
    o_ref[...] = (
        jnp.dot(a_ref[...], w_ref[...], preferred_element_type=jnp.float32)
        + b_ref[...]
    )


_encode = pl.pallas_call(
    _enc_body,
    grid=(E // BE,),
    in_specs=[
        pl.BlockSpec((BE, DE), lambda i: (i, 0)),
        pl.BlockSpec((DE, D), lambda i: (0, 0)),
        pl.BlockSpec((1, D), lambda i: (0, 0)),
    ],
    out_specs=pl.BlockSpec((BE, D), lambda i: (i, 0)),
    out_shape=jax.ShapeDtypeStruct((E, D), jnp.float32),
)


def _layer_body(eps_ref, h_ref, p0_ref, p1_ref, w1_ref, b1_ref, w2_ref,
                b2_ref, o_ref):
    h = h_ref[...]
    t = (1.0 + eps_ref[0]) * h + p0_ref[...] + p1_ref[...]
    t = jnp.maximum(
        jnp.dot(t, w1_ref[...], preferred_element_type=jnp.float32)
        + b1_ref[...], 0.0)
    t = jnp.dot(t, w2_ref[...], preferred_element_type=jnp.float32) + b2_ref[...]
    o_ref[...] = h + jnp.maximum(t, 0.0)


_layer = pl.pallas_call(
    _layer_body,
    grid=(N // BN,),
    in_specs=[
        pl.BlockSpec(memory_space=pltpu.SMEM),
        pl.BlockSpec((BN, D), lambda i: (i, 0)),
        pl.BlockSpec((BN, D), lambda i: (i, 0)),
        pl.BlockSpec((BN, D), lambda i: (i + N // BN, 0)),
        pl.BlockSpec((D, D), lambda i: (0, 0)),
        pl.BlockSpec((1, D), lambda i: (0, 0)),
        pl.BlockSpec((D, D), lambda i: (0, 0)),
        pl.BlockSpec((1, D), lambda i: (0, 0)),
    ],
    out_specs=pl.BlockSpec((BN, D), lambda i: (i, 0)),
    out_shape=jax.ShapeDtypeStruct((N, D), jnp.float32),
)


def _layer_head_body(eps_ref, h_ref, p0_ref, p1_ref, w1_ref, b1_ref, w2_ref,
                     b2_ref, wh_ref, bh_ref, o_ref):
    h = h_ref[...]
    t = (1.0 + eps_ref[0]) * h + p0_ref[...] + p1_ref[...]
    t = jnp.maximum(
        jnp.dot(t, w1_ref[...], preferred_element_type=jnp.float32)
        + b1_ref[...], 0.0)
    t = jnp.dot(t, w2_ref[...], preferred_element_type=jnp.float32) + b2_ref[...]
    h = h + jnp.maximum(t, 0.0)
    o_ref[...] = (
        jnp.dot(h, wh_ref[...], preferred_element_type=jnp.float32)
        + bh_ref[...]
    )


_layer_head = pl.pallas_call(
    _layer_head_body,
    grid=(N // BN,),
    in_specs=[
        pl.BlockSpec(memory_space=pltpu.SMEM),
        pl.BlockSpec((BN, D), lambda i: (i, 0)),
        pl.BlockSpec((BN, D), lambda i: (i, 0)),
        pl.BlockSpec((BN, D), lambda i: (i + N // BN, 0)),
        pl.BlockSpec((D, D), lambda i: (0, 0)),
        pl.BlockSpec((1, D), lambda i: (0, 0)),
        pl.BlockSpec((D, D), lambda i: (0, 0)),
        pl.BlockSpec((1, D), lambda i: (0, 0)),
        pl.BlockSpec((D, D), lambda i: (0, 0)),
        pl.BlockSpec((1, D), lambda i: (0, 0)),
    ],
    out_specs=pl.BlockSpec((BN, D), lambda i: (i, 0)),
    out_shape=jax.ShapeDtypeStruct((N, D_OUT), jnp.float32),
)


def kernel(x, edge_index, edge_attr, W_edge, b_edge, eps, W1s, b1s, W2s, b2s,
           W_head, b_head):
    src = edge_index[0]
    dst = edge_index[1]
    e = _encode(edge_attr, W_edge, b_edge.reshape(1, D))
    h = x
    for l in range(2):
        parts = _sc_aggregate(src, dst, e, h)
        h = _layer(eps[l].reshape(1), h, parts, parts,
                   W1s[l], b1s[l].reshape(1, D),
                   W2s[l], b2s[l].reshape(1, D))
    parts = _sc_aggregate(src, dst, e, h)
    return _layer_head(eps[2].reshape(1), h, parts, parts,
                       W1s[2], b1s[2].reshape(1, D),
                       W2s[2], b2s[2].reshape(1, D),
                       W_head, b_head.reshape(1, D))
